# Initial kernel scaffold; baseline (speedup 1.0000x reference)
import jax
import jax.numpy as jnp
from jax.experimental import pallas as pl


def kernel(h, x, params, edge_index):
    raise NotImplementedError("write your pallas kernel here")


# speedup vs baseline: 2.7543x; 2.7543x over previous
import functools

import jax
import jax.numpy as jnp
from jax import lax
from jax.experimental import pallas as pl
from jax.experimental.pallas import tpu as pltpu
from jax.experimental.pallas import tpu_sc as plsc

N = 10000
E = 320000
DIN = 128
DC = 3
C = 32
NL = 2
EPS = 1e-5

NC = 2
NS = 16
NW = NC * NS
EW = E // NW
BCH = 80
NCH = EW // BCH

f32 = jnp.float32


def _lrelu(v):
    return jnp.where(v >= 0, v, 0.01 * v)


_MESH = plsc.VectorSubcoreMesh(core_axis_name="c", subcore_axis_name="s")
_SC_PARAMS = pltpu.CompilerParams(use_tc_tiling_on_sc=False)


@functools.partial(
    pl.kernel, mesh=_MESH, compiler_params=_SC_PARAMS,
    out_type=jax.ShapeDtypeStruct((4, N), f32),
    scratch_types=[
        pltpu.VMEM((BCH,), jnp.int32),
        pltpu.VMEM((BCH,), f32),
        pltpu.VMEM((N,), f32),
        pltpu.VMEM_SHARED((N,), f32),
        pltpu.VMEM_SHARED((N,), f32),
    ],
)
def _sc_counts(dst_h, src_h, out_h, idx_v, ones_v, z_v, cntd_sh, cnts_sh):
    c = lax.axis_index("c")
    s = lax.axis_index("s")
    wid = c * NS + s

    def fill(i, _):
        ones_v[pl.ds(i * 16, 16)] = jnp.full((16,), 1.0, f32)
        return _
    lax.fori_loop(0, BCH // 16, fill, None)

    def zfill(i, _):
        z_v[pl.ds(i * 16, 16)] = jnp.zeros((16,), f32)
        return _
    lax.fori_loop(0, N // 16, zfill, None)

    @pl.when(s == 0)
    def _():
        pltpu.sync_copy(z_v, cntd_sh)

    @pl.when(s == 1)
    def _():
        pltpu.sync_copy(z_v, cnts_sh)

    plsc.subcore_barrier()

    def chunk(i, _):
        base = wid * EW + i * BCH
        pltpu.sync_copy(dst_h.at[pl.ds(base, BCH)], idx_v)
        pltpu.sync_copy(ones_v, cntd_sh.at[idx_v], add=True)
        pltpu.sync_copy(src_h.at[pl.ds(base, BCH)], idx_v)
        pltpu.sync_copy(ones_v, cnts_sh.at[idx_v], add=True)
        return _
    lax.fori_loop(0, NCH, chunk, None)

    plsc.subcore_barrier()

    @pl.when(s == 0)
    def _():
        pltpu.sync_copy(cntd_sh, out_h.at[c * 2])

    @pl.when(s == 1)
    def _():
        pltpu.sync_copy(cnts_sh, out_h.at[c * 2 + 1])


@functools.partial(
    pl.kernel, mesh=_MESH, compiler_params=_SC_PARAMS,
    out_type=(jax.ShapeDtypeStruct((E, C), f32),
              jax.ShapeDtypeStruct((E, C), f32)),
    scratch_types=[
        pltpu.VMEM((BCH,), jnp.int32),
        pltpu.VMEM((BCH,), jnp.int32),
        pltpu.VMEM((BCH, 2 * C), f32),
        pltpu.VMEM((BCH, 2 * C), f32),
        pltpu.VMEM((BCH, C), f32),
        pltpu.VMEM((BCH, C), f32),
        pltpu.SemaphoreType.DMA,
        pltpu.SemaphoreType.DMA,
    ],
)
def _sc_gather(t_h, u_h, dst_h, src_h, g_out, d_out,
               dstv, srcv, t_v, u_v, g_v, d_v, sem1, sem2):
    c = lax.axis_index("c")
    s = lax.axis_index("s")
    wid = c * NS + s

    def chunk(i, _):
        base = wid * EW + i * BCH
        pltpu.sync_copy(dst_h.at[pl.ds(base, BCH)], dstv)
        pltpu.sync_copy(src_h.at[pl.ds(base, BCH)], srcv)
        cp1 = pltpu.async_copy(t_h.at[dstv], t_v, sem1)
        cp2 = pltpu.async_copy(u_h.at[srcv], u_v, sem2)
        cp1.wait()
        cp2.wait()

        def body(b, _):
            for k in range(2):
                sl = pl.ds(k * 16, 16)
                sh = pl.ds(C + k * 16, 16)
                g_v[b, sl] = t_v[b, sl] + u_v[b, sl]
                d_v[b, sl] = t_v[b, sh] - u_v[b, sh]
            return _
        lax.fori_loop(0, BCH, body, None)

        pltpu.sync_copy(g_v, g_out.at[pl.ds(base, BCH)])
        pltpu.sync_copy(d_v, d_out.at[pl.ds(base, BCH)])
        return _
    lax.fori_loop(0, NCH, chunk, None)


_ZR = 125


@functools.partial(
    pl.kernel, mesh=_MESH, compiler_params=_SC_PARAMS,
    out_type=jax.ShapeDtypeStruct((NC, N, 2 * C), f32),
    scratch_types=[
        pltpu.VMEM((BCH,), jnp.int32),
        pltpu.VMEM((BCH, 2 * C), f32),
        pltpu.VMEM((_ZR, 2 * C), f32),
        pltpu.VMEM_SHARED((N, 2 * C), f32),
    ],
)
def _sc_scatter(feat_h, dst_h, out_h, dstv, f_v, z_v, agg_sh):
    c = lax.axis_index("c")
    s = lax.axis_index("s")
    wid = c * NS + s

    def zfill(i, _):
        for k in range(4):
            z_v[i, pl.ds(k * 16, 16)] = jnp.zeros((16,), f32)
        return _
    lax.fori_loop(0, _ZR, zfill, None)

    row0 = s * (N // NS)
    for j in range(N // NS // _ZR):
        pltpu.sync_copy(z_v, agg_sh.at[pl.ds(row0 + j * _ZR, _ZR)])

    plsc.subcore_barrier()

    def chunk(i, _):
        base = wid * EW + i * BCH
        pltpu.sync_copy(dst_h.at[pl.ds(base, BCH)], dstv)
        pltpu.sync_copy(feat_h.at[pl.ds(base, BCH)], f_v)
        pltpu.sync_copy(f_v, agg_sh.at[dstv], add=True)
        return _
    lax.fori_loop(0, NCH, chunk, None)

    plsc.subcore_barrier()

    for j in range(N // NS // _ZR):
        r0 = row0 + j * _ZR
        pltpu.sync_copy(agg_sh.at[pl.ds(r0, _ZR)], out_h.at[c, pl.ds(r0, _ZR)])


def _readin(h, xp, w1, w2, rb, rg, rbe):
    def body(h_ref, x_ref, w1_ref, w2_ref, rb_ref, rg_ref, rbe_ref,
             hc_ref, xc_ref):
        z = (jnp.dot(h_ref[...], w1_ref[...], preferred_element_type=f32)
             + jnp.dot(x_ref[...], w2_ref[...], preferred_element_type=f32)
             + rb_ref[...])
        mu = jnp.mean(z, axis=0, keepdims=True)
        var = jnp.mean((z - mu) ** 2, axis=0, keepdims=True)
        z = _lrelu(rg_ref[...] * (z - mu) / jnp.sqrt(var + EPS) + rbe_ref[...])
        hc_ref[...] = z[:, :C]
        xc_ref[...] = z[:, C:]

    return pl.pallas_call(
        body,
        out_shape=(jax.ShapeDtypeStruct((N, C), f32),
                   jax.ShapeDtypeStruct((N, C), f32)),
    )(h, xp, w1, w2, rb, rg, rbe)


def _node_pre(hc, xc, cnts, bng, bnb, g64, b64, e1hi, e1hj, e1b):
    def body(hc_ref, xc_ref, cnt_ref, bng_ref, bnb_ref, g64_ref, b64_ref,
             e1hi_ref, e1hj_ref, e1b_ref, y_ref, t_ref, u_ref, bias0_ref):
        hcv = hc_ref[...]
        mu = jnp.mean(hcv, axis=0, keepdims=True)
        var = jnp.mean((hcv - mu) ** 2, axis=0, keepdims=True)
        y = _lrelu(bng_ref[...] * (hcv - mu) / jnp.sqrt(var + EPS)
                   + bnb_ref[...])
        cd = cnt_ref[:, 0:1] + cnt_ref[:, 2:3]
        cs = cnt_ref[:, 1:2] + cnt_ref[:, 3:4]
        y2 = y * y
        s1h = jnp.sum(y * cd, axis=0, keepdims=True)
        s2h = jnp.sum(y2 * cd, axis=0, keepdims=True)
        s1s = jnp.sum(y * cs, axis=0, keepdims=True)
        s2s = jnp.sum(y2 * cs, axis=0, keepdims=True)
        mu_hi = s1h / E
        var_hi = s2h / E - mu_hi * mu_hi
        mu_hj = s1s / E
        var_hj = s2s / E - mu_hj * mu_hj
        g_hi = g64_ref[0:1, :]
        g_hj = g64_ref[1:2, :]
        b_hi = b64_ref[0:1, :]
        b_hj = b64_ref[1:2, :]
        s_hi = g_hi / jnp.sqrt(var_hi + EPS)
        t_hi = b_hi - mu_hi * s_hi
        s_hj = g_hj / jnp.sqrt(var_hj + EPS)
        t_hj = b_hj - mu_hj * s_hj
        a = jnp.dot(y * s_hi, e1hi_ref[...], preferred_element_type=f32)
        b = jnp.dot(y * s_hj, e1hj_ref[...], preferred_element_type=f32)
        bias0_ref[...] = (jnp.dot(t_hi, e1hi_ref[...], preferred_element_type=f32)
                          + jnp.dot(t_hj, e1hj_ref[...], preferred_element_type=f32)
                          + e1b_ref[...])
        y_ref[...] = y
        xcv = xc_ref[...]
        t_ref[:, :C] = a
        t_ref[:, C:] = xcv
        u_ref[:, :C] = b
        u_ref[:, C:] = xcv

    return pl.pallas_call(
        body,
        out_shape=(jax.ShapeDtypeStruct((N, C), f32),
                   jax.ShapeDtypeStruct((N, 2 * C), f32),
                   jax.ShapeDtypeStruct((N, 2 * C), f32),
                   jax.ShapeDtypeStruct((1, C), f32)),
    )(hc, xc, cnts, bng, bnb, g64, b64, e1hi, e1hj, e1b)


_BE = 8000


def _dpass(diff):
    def body(diff_ref, d_ref, s_ref):
        dv = diff_ref[...]
        ssq = jnp.sum(dv * dv, axis=1, keepdims=True)
        d = jnp.sqrt(ssq)
        d_ref[...] = d
        lane = lax.broadcasted_iota(jnp.int32, (1, 8), 1)
        contrib = (jnp.where(lane == 0, jnp.sum(d), 0.0)
                   + jnp.where(lane == 1, jnp.sum(ssq), 0.0))

        @pl.when(pl.program_id(0) == 0)
        def _():
            s_ref[...] = jnp.zeros((1, 8), f32)

        s_ref[...] += contrib

    return pl.pallas_call(
        body,
        grid=(E // _BE,),
        in_specs=[pl.BlockSpec((_BE, C), lambda i: (i, 0))],
        out_specs=(pl.BlockSpec((_BE, 1), lambda i: (i, 0)),
                   pl.BlockSpec((1, 8), lambda i: (0, 0))),
        out_shape=(jax.ShapeDtypeStruct((E, 1), f32),
                   jax.ShapeDtypeStruct((1, 8), f32)),
    )(diff)


def _edge_mlp(g, diff, d, coef, e1d_bias0, e2w, e2b, c1w, c1b, c2w):
    def body(g_ref, diff_ref, d_ref, coef_ref, eb_ref, e2w_ref, e2b_ref,
             c1w_ref, c1b_ref, c2w_ref, feat_ref):
        sd = coef_ref[0:1, 0:1]
        td = coef_ref[0:1, 1:2]
        e1d = eb_ref[0:1, :]
        bias0 = eb_ref[1:2, :]
        dq = d_ref[...] * sd + td
        pre = g_ref[...] + dq * e1d + bias0
        m = _lrelu(pre)
        m = _lrelu(jnp.dot(m, e2w_ref[...], preferred_element_type=f32)
                   + e2b_ref[...])
        cm = jnp.dot(
            _lrelu(jnp.dot(m, c1w_ref[...], preferred_element_type=f32)
                   + c1b_ref[...]),
            c2w_ref[...], preferred_element_type=f32)
        feat_ref[:, :C] = m
        feat_ref[:, C:] = diff_ref[...] * cm

    return pl.pallas_call(
        body,
        grid=(E // _BE,),
        in_specs=[
            pl.BlockSpec((_BE, C), lambda i: (i, 0)),
            pl.BlockSpec((_BE, C), lambda i: (i, 0)),
            pl.BlockSpec((_BE, 1), lambda i: (i, 0)),
            pl.BlockSpec((1, 8), lambda i: (0, 0)),
            pl.BlockSpec((2, C), lambda i: (0, 0)),
            pl.BlockSpec((C, C), lambda i: (0, 0)),
            pl.BlockSpec((1, C), lambda i: (0, 0)),
            pl.BlockSpec((C, C), lambda i: (0, 0)),
            pl.BlockSpec((1, C), lambda i: (0, 0)),
            pl.BlockSpec((C, C), lambda i: (0, 0)),
        ],
        out_specs=pl.BlockSpec((_BE, 2 * C), lambda i: (i, 0)),
        out_shape=jax.ShapeDtypeStruct((E, 2 * C), f32),
    )(g, diff, d, coef, e1d_bias0, e2w, e2b, c1w, c1b, c2w)


def _node_post(agg0, agg1, y, hc, n1a, n1b, n1bias, n2w, n2b):
    def body(a0_ref, a1_ref, y_ref, hc_ref, n1a_ref, n1b_ref, n1bias_ref,
             n2w_ref, n2b_ref, hco_ref, xco_ref):
        agg = a0_ref[...] + a1_ref[...]
        magg = agg[:, :C]
        xn = agg[:, C:]
        hn = _lrelu(jnp.dot(y_ref[...], n1a_ref[...], preferred_element_type=f32)
                    + jnp.dot(magg, n1b_ref[...], preferred_element_type=f32)
                    + n1bias_ref[...])
        hn = jnp.dot(hn, n2w_ref[...], preferred_element_type=f32) + n2b_ref[...]
        hco_ref[...] = hc_ref[...] + hn
        xco_ref[...] = xn

    return pl.pallas_call(
        body,
        out_shape=(jax.ShapeDtypeStruct((N, C), f32),
                   jax.ShapeDtypeStruct((N, C), f32)),
    )(agg0, agg1, y, hc, n1a, n1b, n1bias, n2w, n2b)


def _readout(hc, row, rob):
    def body(hc_ref, w_ref, b_ref, o_ref):
        o_ref[...] = (jnp.dot(hc_ref[...], w_ref[...],
                              preferred_element_type=f32) + b_ref[...])

    return pl.pallas_call(
        body,
        out_shape=jax.ShapeDtypeStruct((N, DIN), f32),
    )(hc, row, rob)


def kernel(h, x, params, edge_index):
    src = edge_index[0]
    dst = edge_index[1]

    cntp = _sc_counts(dst, src)
    cnts = jnp.transpose(cntp)

    xp = jnp.pad(x, ((0, 0), (0, 8 - DC)))
    w2 = jnp.pad(params['ri_W'][DIN:], ((0, 8 - DC), (0, 0)))
    hc, xc = _readin(h, xp, params['ri_W'][:DIN], w2,
                     params['ri_b'][None, :], params['ri_g'][None, :],
                     params['ri_be'][None, :])

    for l in range(NL):
        p = params['l%d' % l]
        g64 = jnp.stack([p['ein_g'][:C], p['ein_g'][C:2 * C]])
        b64 = jnp.stack([p['ein_b'][:C], p['ein_b'][C:2 * C]])
        y, t, u, bias0 = _node_pre(
            hc, xc, cnts, p['bn_g'][None, :], p['bn_b'][None, :], g64, b64,
            p['e1_W'][:C], p['e1_W'][C:2 * C], p['e1_b'][None, :])

        g, diff = _sc_gather(t, u, dst, src)

        d, sums = _dpass(diff)
        mu_d = sums[0, 0] / E
        var_d = sums[0, 1] / E - mu_d * mu_d
        s_d = p['ein_g'][2 * C] / jnp.sqrt(var_d + EPS)
        t_d = p['ein_b'][2 * C] - mu_d * s_d
        coef = jnp.zeros((1, 8), f32).at[0, 0].set(s_d).at[0, 1].set(t_d)
        e1d_bias0 = jnp.concatenate([p['e1_W'][2 * C][None, :], bias0], axis=0)

        feat = _edge_mlp(g, diff, d, coef, e1d_bias0,
                         p['e2_W'], p['e2_b'][None, :],
                         p['c1_W'], p['c1_b'][None, :], p['c2_W'])

        aggp = _sc_scatter(feat, dst)

        hc, xc = _node_post(aggp[0], aggp[1], y, hc,
                            p['n1_W'][:C], p['n1_W'][C:],
                            p['n1_b'][None, :], p['n2_W'],
                            p['n2_b'][None, :])

    return _readout(hc, params['ro_W'], params['ro_b'][None, :])
